# baseline (speedup 1.0000x reference)
import functools

import jax
import jax.numpy as jnp
from jax import lax
from jax.experimental import pallas as pl
from jax.experimental.pallas import tpu as pltpu
from jax.experimental.pallas import tpu_sc as plsc

N = 10000
E = 320000
D = 128
H = 8
K = 16
F = 512

NC = 2
NS = 16
NT = NC * NS
EPT = E // NT
WW = 80
NW = EPT // WW
RPT = N // NS
VPW = WW // 16

_MESH = dict(core_axis_name="c", subcore_axis_name="s", num_cores=NC,
             num_subcores=NS)


def _lane():
  return lax.iota(jnp.int32, 16)


def _c16(v):
  return jnp.full((16,), v, jnp.int32)


def _proj_body(w_ref, wf_ref, ab_ref, wh_ref, sc_ref):
  wh = jnp.dot(w_ref[...], wf_ref[...], preferred_element_type=jnp.float32)
  wh_ref[...] = wh
  sc_ref[...] = jnp.dot(wh, ab_ref[...], preferred_element_type=jnp.float32)


def _project(w, wflat, abig):
  nb = 5
  bn = N // nb
  return pl.pallas_call(
      _proj_body,
      grid=(nb,),
      in_specs=[
          pl.BlockSpec((bn, D), lambda i: (i, 0)),
          pl.BlockSpec((D, D), lambda i: (0, 0)),
          pl.BlockSpec((D, 2 * H), lambda i: (0, 0)),
      ],
      out_specs=[
          pl.BlockSpec((bn, D), lambda i: (i, 0)),
          pl.BlockSpec((bn, 2 * H), lambda i: (i, 0)),
      ],
      out_shape=[
          jax.ShapeDtypeStruct((N, D), jnp.float32),
          jax.ShapeDtypeStruct((N, 2 * H), jnp.float32),
      ],
  )(w, wflat, abig)


def _segmax_body(scores_hbm, srcslab_hbm, dstflat_hbm, p_out,
                 p_v, srcidx_v, dstflat_v, rows_v, sem):
  t = lax.axis_index("s") * NC + lax.axis_index("c")
  pltpu.sync_copy(srcslab_hbm.at[t], srcidx_v)
  pltpu.sync_copy(dstflat_hbm.at[t], dstflat_v)

  def init(i, _):
    p_v[pl.ds(i * 16, 16)] = jnp.full((16,), -1e30, jnp.float32)
    return 0
  lax.fori_loop(0, (N * H) // 16, init, 0)

  lane = _lane()
  lane8 = lane & 7
  mask8 = lane < H

  def win(wi, _):
    pltpu.async_copy(scores_hbm.at[srcidx_v.at[wi]], rows_v, sem).wait()

    def edge(ei, _):
      srow = rows_v[ei, :]
      dstv = plsc.load_gather(dstflat_v, [_c16(wi * WW + ei)])
      adr = dstv * H + lane8
      old = plsc.load_gather(p_v, [adr])
      plsc.store_scatter(p_v, [adr], jnp.maximum(old, srow), mask=mask8)
      return 0
    lax.fori_loop(0, WW, edge, 0)
    return 0
  lax.fori_loop(0, NW, win, 0)

  pltpu.sync_copy(p_v, p_out.at[t])


def _segmax(scores, srcslab, dstflat):
  fn = pl.kernel(
      _segmax_body,
      out_type=jax.ShapeDtypeStruct((NT, N * H), jnp.float32),
      mesh=plsc.VectorSubcoreMesh(**_MESH),
      compiler_params=pltpu.CompilerParams(needs_layout_passes=False, use_tc_tiling_on_sc=False),
      scratch_types=[
          pltpu.VMEM((N * H,), jnp.float32),
          pltpu.VMEM((NW, WW), jnp.int32),
          pltpu.VMEM((EPT,), jnp.int32),
          pltpu.VMEM((WW, 2 * H), jnp.float32),
          pltpu.SemaphoreType.DMA,
      ],
  )
  return fn(scores, srcslab, dstflat)


def _pmax_body(p_ref, out_ref):
  out_ref[...] = jnp.max(p_ref[...], axis=0, keepdims=True)


def _pmax(p_part):
  nb = 5
  bc = (N * H) // nb
  out = pl.pallas_call(
      _pmax_body,
      grid=(nb,),
      in_specs=[pl.BlockSpec((NT, bc), lambda i: (0, i))],
      out_specs=pl.BlockSpec((1, bc), lambda i: (0, i)),
      out_shape=jax.ShapeDtypeStruct((1, N * H), jnp.float32),
  )(p_part)
  return out.reshape(N, H)


def _edm_body(p_ref, sc_ref, out_ref):
  ed = sc_ref[:, H:]
  z = p_ref[...] + ed
  m = jnp.where(z > 0, z, 0.2 * z)
  out_ref[...] = jnp.concatenate([ed, m], axis=1)


def _edm(p2, scores):
  nb = 5
  bn = N // nb
  return pl.pallas_call(
      _edm_body,
      grid=(nb,),
      in_specs=[
          pl.BlockSpec((bn, H), lambda i: (i, 0)),
          pl.BlockSpec((bn, 2 * H), lambda i: (i, 0)),
      ],
      out_specs=pl.BlockSpec((bn, 2 * H), lambda i: (i, 0)),
      out_shape=jax.ShapeDtypeStruct((N, 2 * H), jnp.float32),
  )(p2, scores)


def _edge_body(wh_hbm, scores_hbm, edm_hbm, zero128_hbm, zero16_hbm,
               srcslab_hbm, dstflat_hbm,
               msg_out, den_out,
               srcidx_v, dstflat_v, dstmod_v, srcrows_v, edmrows_v,
               whbuf_v, exbuf_v, msg_s, den_s, sem1, sem2, sem3):
  cid = lax.axis_index("c")
  sid = lax.axis_index("s")
  t = sid * NC + cid
  pltpu.sync_copy(srcslab_hbm.at[t], srcidx_v)
  pltpu.sync_copy(dstflat_hbm.at[t], dstflat_v)
  pltpu.sync_copy(zero128_hbm.at[pl.ds(sid * RPT, RPT)],
                  msg_s.at[pl.ds(sid * RPT, RPT)])
  pltpu.sync_copy(zero16_hbm.at[pl.ds(sid * RPT, RPT)],
                  den_s.at[pl.ds(sid * RPT, RPT)])
  plsc.subcore_barrier()

  lane = _lane()
  midx = (lane & 7) + H
  mask8 = lane < H

  mask1 = lane < 1

  def win(wi, _):
    c2 = pltpu.async_copy(edm_hbm.at[dstflat_v.at[pl.ds(wi * WW, WW)]],
                          edmrows_v, sem2)
    c1 = pltpu.async_copy(scores_hbm.at[srcidx_v.at[wi]], srcrows_v, sem1)
    c3 = pltpu.async_copy(wh_hbm.at[srcidx_v.at[wi]], whbuf_v, sem3)
    c2.wait()
    c1.wait()
    c3.wait()

    def edge(ei, _):
      srow = srcrows_v[ei, :]
      erow = edmrows_v[ei, :]
      mvec = erow.at[midx].get(mode="promise_in_bounds")
      z = srow + erow
      e = jnp.where(z > 0, z, 0.2 * z)
      ex = jnp.where(mask8, jnp.exp(jnp.minimum(e - mvec, 0.0)), 0.0)
      exbuf_v[ei, :] = ex
      nz = jnp.max(ex) > 0.0
      dstv = plsc.load_gather(dstflat_v, [_c16(wi * WW + ei)])
      sel = jnp.where(jnp.full((16,), nz), dstv, _c16(N) + (_c16(ei) & 7))
      plsc.store_scatter(dstmod_v, [_c16(ei)], sel, mask=mask1)

      @pl.when(nz)
      def _():
        for h in range(H):
          exh = ex.at[_c16(h)].get(mode="promise_in_bounds")
          seg = whbuf_v[ei, pl.ds(h * K, K)]
          whbuf_v[ei, pl.ds(h * K, K)] = seg * exh
      return 0
    lax.fori_loop(0, WW, edge, 0)

    pltpu.sync_copy(whbuf_v, msg_s.at[dstmod_v], add=True)
    pltpu.sync_copy(exbuf_v, den_s.at[dstmod_v], add=True)
    return 0
  lax.fori_loop(0, NW, win, 0)

  plsc.subcore_barrier()
  pltpu.sync_copy(msg_s.at[pl.ds(sid * RPT, RPT)],
                  msg_out.at[cid, pl.ds(sid * RPT, RPT)])
  pltpu.sync_copy(den_s.at[pl.ds(sid * RPT, RPT)],
                  den_out.at[cid, pl.ds(sid * RPT, RPT)])


def _edge_phase(wh, scores, edm, zero128, zero16, srcslab, dstflat):
  fn = pl.kernel(
      _edge_body,
      out_type=(
          jax.ShapeDtypeStruct((NC, N, D), jnp.float32),
          jax.ShapeDtypeStruct((NC, N, 2 * H), jnp.float32),
      ),
      mesh=plsc.VectorSubcoreMesh(**_MESH),
      compiler_params=pltpu.CompilerParams(needs_layout_passes=False, use_tc_tiling_on_sc=False),
      scratch_types=[
          pltpu.VMEM((NW, WW), jnp.int32),
          pltpu.VMEM((EPT,), jnp.int32),
          pltpu.VMEM((WW,), jnp.int32),
          pltpu.VMEM((WW, 2 * H), jnp.float32),
          pltpu.VMEM((WW, 2 * H), jnp.float32),
          pltpu.VMEM((WW, D), jnp.float32),
          pltpu.VMEM((WW, 2 * H), jnp.float32),
          pltpu.VMEM_SHARED((N + 8, D), jnp.float32),
          pltpu.VMEM_SHARED((N + 8, 2 * H), jnp.float32),
          pltpu.SemaphoreType.DMA,
          pltpu.SemaphoreType.DMA,
          pltpu.SemaphoreType.DMA,
      ],
  )
  return fn(wh, scores, edm, zero128, zero16, srcslab, dstflat)


def _post_body(mp_ref, dp_ref, s_ref, erep_ref, lng_ref, lnb_ref,
               w1_ref, b1_ref, w2_ref, b2_ref, out_ref):
  acc = mp_ref[0] + mp_ref[1]
  den = dp_ref[0, :, :H] + dp_ref[1, :, :H]
  recip = 1.0 / (den + 1e-10)
  gat = acc * jnp.dot(recip, erep_ref[...], preferred_element_type=jnp.float32)
  g = jnp.where(gat > 0, gat, jnp.exp(jnp.minimum(gat, 0.0)) - 1.0)
  hh = g + s_ref[...]
  mu = jnp.mean(hh, axis=-1, keepdims=True)
  xc = hh - mu
  var = jnp.mean(xc * xc, axis=-1, keepdims=True)
  hn = xc / jnp.sqrt(var + 1e-6) * lng_ref[...] + lnb_ref[...]
  inter = jnp.dot(hn, w1_ref[...], preferred_element_type=jnp.float32)
  inter = jnp.maximum(inter + b1_ref[...], 0.0)
  out = jnp.dot(inter, w2_ref[...], preferred_element_type=jnp.float32)
  out_ref[...] = out + b2_ref[...] + hh


def _post(msg_part, den_part, s, erep, ln_g, ln_b, w1, b1, w2, b2):
  nb = 5
  bn = N // nb
  return pl.pallas_call(
      _post_body,
      grid=(nb,),
      in_specs=[
          pl.BlockSpec((NC, bn, D), lambda i: (0, i, 0)),
          pl.BlockSpec((NC, bn, 2 * H), lambda i: (0, i, 0)),
          pl.BlockSpec((bn, D), lambda i: (i, 0)),
          pl.BlockSpec((H, D), lambda i: (0, 0)),
          pl.BlockSpec((1, D), lambda i: (0, 0)),
          pl.BlockSpec((1, D), lambda i: (0, 0)),
          pl.BlockSpec((D, F), lambda i: (0, 0)),
          pl.BlockSpec((1, F), lambda i: (0, 0)),
          pl.BlockSpec((F, D), lambda i: (0, 0)),
          pl.BlockSpec((1, D), lambda i: (0, 0)),
      ],
      out_specs=pl.BlockSpec((bn, D), lambda i: (i, 0)),
      out_shape=jax.ShapeDtypeStruct((N, D), jnp.float32),
  )(msg_part, den_part, s, erep, ln_g, ln_b, w1, b1, w2, b2)


def kernel(w, s, edge_index, W, a_src, a_dst, ln_g, ln_b, W1, b1, W2, b2):
  w = w.astype(jnp.float32)
  src = edge_index[0].astype(jnp.int32)
  dst = edge_index[1].astype(jnp.int32)

  wflat = jnp.transpose(W, (1, 0, 2)).reshape(D, H * K)
  col = jnp.arange(D)
  hcol = col // K
  asrc_m = jnp.zeros((D, H), jnp.float32).at[col, hcol].set(a_src.reshape(-1))
  adst_m = jnp.zeros((D, H), jnp.float32).at[col, hcol].set(a_dst.reshape(-1))
  abig = jnp.concatenate([asrc_m, adst_m], axis=1)
  erep = jnp.repeat(jnp.eye(H, dtype=jnp.float32), K, axis=1)

  srcslab = src.reshape(NT, NW, WW)
  dstflat = dst.reshape(NT, EPT)
  zero128 = jnp.zeros((N, D), jnp.float32)
  zero16 = jnp.zeros((N, 2 * H), jnp.float32)

  wh, scores = _project(w, wflat, abig)
  p_part = _segmax(scores, srcslab, dstflat)
  edm = _edm(_pmax(p_part), scores)
  msg_part, den_part = _edge_phase(wh, scores, edm, zero128, zero16,
                                   srcslab, dstflat)
  return _post(msg_part, den_part, s, erep,
               ln_g.reshape(1, D), ln_b.reshape(1, D),
               W1, b1.reshape(1, F), W2, b2.reshape(1, D))


# speedup vs baseline: 1.2927x; 1.2927x over previous
import functools

import jax
import jax.numpy as jnp
from jax import lax
from jax.experimental import pallas as pl
from jax.experimental.pallas import tpu as pltpu
from jax.experimental.pallas import tpu_sc as plsc

N = 10000
E = 320000
D = 128
H = 8
K = 16
F = 512

NC = 2
NS = 16
NT = NC * NS
EPT = E // NT
WW = 80
NW = EPT // WW
RPT = N // NS
VPW = WW // 16

_MESH = dict(core_axis_name="c", subcore_axis_name="s", num_cores=NC,
             num_subcores=NS)


def _lane():
  return lax.iota(jnp.int32, 16)


def _c16(v):
  return jnp.full((16,), v, jnp.int32)


def _proj_body(w_ref, wf_ref, ab_ref, wha_ref, whb_ref, sc_ref):
  wh = jnp.dot(w_ref[...], wf_ref[...], preferred_element_type=jnp.float32)
  wha_ref[...] = wh[:, :D // 2]
  whb_ref[...] = wh[:, D // 2:]
  sc_ref[...] = jnp.dot(wh, ab_ref[...], preferred_element_type=jnp.float32)


def _project(w, wflat, abig):
  nb = 5
  bn = N // nb
  return pl.pallas_call(
      _proj_body,
      grid=(nb,),
      in_specs=[
          pl.BlockSpec((bn, D), lambda i: (i, 0)),
          pl.BlockSpec((D, D), lambda i: (0, 0)),
          pl.BlockSpec((D, 2 * H), lambda i: (0, 0)),
      ],
      out_specs=[
          pl.BlockSpec((bn, D // 2), lambda i: (i, 0)),
          pl.BlockSpec((bn, D // 2), lambda i: (i, 0)),
          pl.BlockSpec((bn, 2 * H), lambda i: (i, 0)),
      ],
      out_shape=[
          jax.ShapeDtypeStruct((N, D // 2), jnp.float32),
          jax.ShapeDtypeStruct((N, D // 2), jnp.float32),
          jax.ShapeDtypeStruct((N, 2 * H), jnp.float32),
      ],
  )(w, wflat, abig)


def _segmax_body(scores_hbm, srcslab_hbm, dstflat_hbm, p_out,
                 p_v, srcidx_v, dstflat_v, rows_v, sem):
  t = lax.axis_index("s") * NC + lax.axis_index("c")
  pltpu.sync_copy(srcslab_hbm.at[t], srcidx_v)
  pltpu.sync_copy(dstflat_hbm.at[t], dstflat_v)

  def init(i, _):
    p_v[pl.ds(i * 16, 16)] = jnp.full((16,), -1e30, jnp.float32)
    return 0
  lax.fori_loop(0, (N * H) // 16, init, 0)

  lane = _lane()
  lane8 = lane & 7
  mask8 = lane < H

  def win(wi, _):
    pltpu.async_copy(scores_hbm.at[srcidx_v.at[wi]], rows_v, sem).wait()

    def edge(ei, _):
      srow = rows_v[ei, :]
      dstv = plsc.load_gather(dstflat_v, [_c16(wi * WW + ei)])
      adr = dstv * H + lane8
      old = plsc.load_gather(p_v, [adr])
      plsc.store_scatter(p_v, [adr], jnp.maximum(old, srow), mask=mask8)
      return 0
    lax.fori_loop(0, WW, edge, 0)
    return 0
  lax.fori_loop(0, NW, win, 0)

  pltpu.sync_copy(p_v, p_out.at[t])


def _segmax(scores, srcslab, dstflat):
  fn = pl.kernel(
      _segmax_body,
      out_type=jax.ShapeDtypeStruct((NT, N * H), jnp.float32),
      mesh=plsc.VectorSubcoreMesh(**_MESH),
      compiler_params=pltpu.CompilerParams(needs_layout_passes=False, use_tc_tiling_on_sc=False),
      scratch_types=[
          pltpu.VMEM((N * H,), jnp.float32),
          pltpu.VMEM((NW, WW), jnp.int32),
          pltpu.VMEM((EPT,), jnp.int32),
          pltpu.VMEM((WW, 2 * H), jnp.float32),
          pltpu.SemaphoreType.DMA,
      ],
  )
  return fn(scores, srcslab, dstflat)


def _pmax_body(p_ref, out_ref):
  out_ref[...] = jnp.max(p_ref[...], axis=0, keepdims=True)


def _pmax(p_part):
  nb = 5
  bc = (N * H) // nb
  out = pl.pallas_call(
      _pmax_body,
      grid=(nb,),
      in_specs=[pl.BlockSpec((NT, bc), lambda i: (0, i))],
      out_specs=pl.BlockSpec((1, bc), lambda i: (0, i)),
      out_shape=jax.ShapeDtypeStruct((1, N * H), jnp.float32),
  )(p_part)
  return out.reshape(N, H)


def _edm_body(p_ref, sc_ref, out_ref):
  ed = sc_ref[:, H:]
  z = p_ref[...] + ed
  m = jnp.where(z > 0, z, 0.2 * z)
  out_ref[...] = jnp.concatenate([ed, m], axis=1)


def _edm(p2, scores):
  nb = 5
  bn = N // nb
  return pl.pallas_call(
      _edm_body,
      grid=(nb,),
      in_specs=[
          pl.BlockSpec((bn, H), lambda i: (i, 0)),
          pl.BlockSpec((bn, 2 * H), lambda i: (i, 0)),
      ],
      out_specs=pl.BlockSpec((bn, 2 * H), lambda i: (i, 0)),
      out_shape=jax.ShapeDtypeStruct((N, 2 * H), jnp.float32),
  )(p2, scores)


CH = 128
MAXNZ = EPT + CH


def _edge_body(wha_hbm, whb_hbm, scores_hbm, edm_hbm, zero64_hbm, zero16_hbm,
               srcslab_hbm, dstslabr_hbm, sdflat_hbm,
               msga_out, msgb_out, den_out,
               srcidx_v, dstidxr_v, sdflat_v, nzid_v,
               srclist_v, dstlist_v,
               srcrows_v, edmrows_v, whc_v, sc2_v, ed2_v, exbuf_v,
               msg_s, den_s, sem1, sem2, sem3):
  cid = lax.axis_index("c")
  sid = lax.axis_index("s")
  t = sid * NC + cid
  pltpu.sync_copy(srcslab_hbm.at[t], srcidx_v)
  pltpu.sync_copy(dstslabr_hbm.at[t], dstidxr_v)
  pltpu.sync_copy(sdflat_hbm.at[t], sdflat_v)
  pltpu.sync_copy(zero16_hbm, den_s.at[pl.ds(sid * RPT, RPT)])
  plsc.subcore_barrier()

  lane = _lane()
  midx = (lane & 7) + H
  mask8 = lane < H

  mask1 = lane < 1

  def win(wi, ctr):
    c2 = pltpu.async_copy(edm_hbm.at[dstidxr_v.at[wi]], edmrows_v, sem2)
    c1 = pltpu.async_copy(scores_hbm.at[srcidx_v.at[wi]], srcrows_v, sem1)
    c2.wait()
    c1.wait()

    def edge(ei, ctr):
      srow = srcrows_v[ei, :]
      erow = edmrows_v[ei, :]
      mvec = erow.at[midx].get(mode="promise_in_bounds")
      z = srow + erow
      e = jnp.where(z > 0, z, 0.2 * z)
      ex = jnp.where(mask8, jnp.exp(jnp.where(mask8, e - mvec, -1e30)), 0.0)
      exbuf_v[ei, :] = ex
      plsc.store_scatter(nzid_v, [_c16(ctr)], _c16(wi * WW + ei), mask=mask1)
      nzc = plsc.all_reduce_population_count(ex > 0.0)
      return ctr + jnp.minimum(nzc[0], 1)
    ctr = lax.fori_loop(0, WW, edge, ctr)

    pltpu.sync_copy(exbuf_v, den_s.at[dstidxr_v.at[wi]], add=True)
    return ctr
  cnt = lax.fori_loop(0, NW, win, 0)

  plsc.subcore_barrier()
  pltpu.sync_copy(den_s.at[pl.ds(sid * RPT, RPT)],
                  den_out.at[cid, pl.ds(sid * RPT, RPT)])

  nchunks = (cnt + CH - 1) // CH

  for hp, (wh_hbm, m_out) in enumerate(((wha_hbm, msga_out),
                                        (whb_hbm, msgb_out))):
    pltpu.sync_copy(zero64_hbm, msg_s.at[pl.ds(sid * RPT, RPT)])
    plsc.subcore_barrier()

    def chunk(c, _):
      for v in range(CH // 16):
        j16 = _c16(c * CH + v * 16) + lane
        valid = j16 < _c16(cnt)
        eids = nzid_v[pl.ds(c * CH + v * 16, 16)]
        eids = jnp.where(valid, eids, _c16(0))
        srcs = plsc.load_gather(sdflat_v, [eids])
        dsts = plsc.load_gather(sdflat_v, [eids + _c16(EPT)])
        dsts = jnp.where(valid, dsts, _c16(N) + (j16 & 7))
        srclist_v[pl.ds(v * 16, 16)] = srcs
        dstlist_v[pl.ds(v * 16, 16)] = dsts
      g1 = pltpu.async_copy(wh_hbm.at[srclist_v], whc_v, sem3)
      g2 = pltpu.async_copy(scores_hbm.at[srclist_v], sc2_v, sem1)
      g3 = pltpu.async_copy(edm_hbm.at[dstlist_v], ed2_v, sem2)
      g1.wait()
      g2.wait()
      g3.wait()

      def row(r, _):
        srow = sc2_v[r, :]
        erow = ed2_v[r, :]
        mvec = erow.at[midx].get(mode="promise_in_bounds")
        z = srow + erow
        e = jnp.where(z > 0, z, 0.2 * z)
        ex = jnp.where(mask8, jnp.exp(jnp.where(mask8, e - mvec, -1e30)), 0.0)
        for h in range(H // 2):
          exh = ex.at[_c16(hp * (H // 2) + h)].get(mode="promise_in_bounds")
          seg = whc_v[r, pl.ds(h * K, K)]
          whc_v[r, pl.ds(h * K, K)] = seg * exh
        return 0
      lax.fori_loop(0, CH, row, 0)

      pltpu.sync_copy(whc_v, msg_s.at[dstlist_v], add=True)
      return 0
    lax.fori_loop(0, nchunks, chunk, 0)

    plsc.subcore_barrier()
    pltpu.sync_copy(msg_s.at[pl.ds(sid * RPT, RPT)],
                    m_out.at[cid, pl.ds(sid * RPT, RPT)])
    plsc.subcore_barrier()


def _edge_phase(wha, whb, scores, edm, zero64, zero16, srcslab, dstslabr,
                sdflat):
  fn = pl.kernel(
      _edge_body,
      out_type=(
          jax.ShapeDtypeStruct((NC, N, D // 2), jnp.float32),
          jax.ShapeDtypeStruct((NC, N, D // 2), jnp.float32),
          jax.ShapeDtypeStruct((NC, N, 2 * H), jnp.float32),
      ),
      mesh=plsc.VectorSubcoreMesh(**_MESH),
      compiler_params=pltpu.CompilerParams(needs_layout_passes=False, use_tc_tiling_on_sc=False),
      scratch_types=[
          pltpu.VMEM((NW, WW), jnp.int32),
          pltpu.VMEM((NW, WW), jnp.int32),
          pltpu.VMEM((2 * EPT,), jnp.int32),
          pltpu.VMEM((MAXNZ,), jnp.int32),
          pltpu.VMEM((CH,), jnp.int32),
          pltpu.VMEM((CH,), jnp.int32),
          pltpu.VMEM((WW, 2 * H), jnp.float32),
          pltpu.VMEM((WW, 2 * H), jnp.float32),
          pltpu.VMEM((CH, D // 2), jnp.float32),
          pltpu.VMEM((CH, 2 * H), jnp.float32),
          pltpu.VMEM((CH, 2 * H), jnp.float32),
          pltpu.VMEM((WW, 2 * H), jnp.float32),
          pltpu.VMEM_SHARED((N + 8, D // 2), jnp.float32),
          pltpu.VMEM_SHARED((N + 8, 2 * H), jnp.float32),
          pltpu.SemaphoreType.DMA,
          pltpu.SemaphoreType.DMA,
          pltpu.SemaphoreType.DMA,
      ],
  )
  return fn(wha, whb, scores, edm, zero64, zero16, srcslab, dstslabr,
            sdflat)


def _post_body(ma_ref, mb_ref, dp_ref, s_ref, erep_ref, lng_ref, lnb_ref,
               w1_ref, b1_ref, w2_ref, b2_ref, out_ref):
  acc = jnp.concatenate([ma_ref[0] + ma_ref[1], mb_ref[0] + mb_ref[1]],
                        axis=1)
  den = dp_ref[0, :, :H] + dp_ref[1, :, :H]
  recip = 1.0 / (den + 1e-10)
  gat = acc * jnp.dot(recip, erep_ref[...], preferred_element_type=jnp.float32)
  g = jnp.where(gat > 0, gat, jnp.exp(jnp.minimum(gat, 0.0)) - 1.0)
  hh = g + s_ref[...]
  mu = jnp.mean(hh, axis=-1, keepdims=True)
  xc = hh - mu
  var = jnp.mean(xc * xc, axis=-1, keepdims=True)
  hn = xc / jnp.sqrt(var + 1e-6) * lng_ref[...] + lnb_ref[...]
  inter = jnp.dot(hn, w1_ref[...], preferred_element_type=jnp.float32)
  inter = jnp.maximum(inter + b1_ref[...], 0.0)
  out = jnp.dot(inter, w2_ref[...], preferred_element_type=jnp.float32)
  out_ref[...] = out + b2_ref[...] + hh


def _post(msga, msgb, den_part, s, erep, ln_g, ln_b, w1, b1, w2, b2):
  nb = 5
  bn = N // nb
  return pl.pallas_call(
      _post_body,
      grid=(nb,),
      in_specs=[
          pl.BlockSpec((NC, bn, D // 2), lambda i: (0, i, 0)),
          pl.BlockSpec((NC, bn, D // 2), lambda i: (0, i, 0)),
          pl.BlockSpec((NC, bn, 2 * H), lambda i: (0, i, 0)),
          pl.BlockSpec((bn, D), lambda i: (i, 0)),
          pl.BlockSpec((H, D), lambda i: (0, 0)),
          pl.BlockSpec((1, D), lambda i: (0, 0)),
          pl.BlockSpec((1, D), lambda i: (0, 0)),
          pl.BlockSpec((D, F), lambda i: (0, 0)),
          pl.BlockSpec((1, F), lambda i: (0, 0)),
          pl.BlockSpec((F, D), lambda i: (0, 0)),
          pl.BlockSpec((1, D), lambda i: (0, 0)),
      ],
      out_specs=pl.BlockSpec((bn, D), lambda i: (i, 0)),
      out_shape=jax.ShapeDtypeStruct((N, D), jnp.float32),
  )(msga, msgb, den_part, s, erep, ln_g, ln_b, w1, b1, w2, b2)


def kernel(w, s, edge_index, W, a_src, a_dst, ln_g, ln_b, W1, b1, W2, b2):
  w = w.astype(jnp.float32)
  src = edge_index[0].astype(jnp.int32)
  dst = edge_index[1].astype(jnp.int32)

  wflat = jnp.transpose(W, (1, 0, 2)).reshape(D, H * K)
  col = jnp.arange(D)
  hcol = col // K
  asrc_m = jnp.zeros((D, H), jnp.float32).at[col, hcol].set(a_src.reshape(-1))
  adst_m = jnp.zeros((D, H), jnp.float32).at[col, hcol].set(a_dst.reshape(-1))
  abig = jnp.concatenate([asrc_m, adst_m], axis=1)
  erep = jnp.repeat(jnp.eye(H, dtype=jnp.float32), K, axis=1)

  srcslab = src.reshape(NT, NW, WW)
  dstslabr = dst.reshape(NT, NW, WW)
  dstflat = dst.reshape(NT, EPT)
  sdflat = jnp.concatenate([src.reshape(NT, EPT), dst.reshape(NT, EPT)],
                           axis=1)
  zero64 = jnp.zeros((RPT, D // 2), jnp.float32)
  zero16 = jnp.zeros((RPT, 2 * H), jnp.float32)

  wha, whb, scores = _project(w, wflat, abig)
  p_part = _segmax(scores, srcslab, dstflat)
  edm = _edm(_pmax(p_part), scores)
  msga, msgb, den_part = _edge_phase(wha, whb, scores, edm, zero64, zero16,
                                     srcslab, dstslabr, sdflat)
  return _post(msga, msgb, den_part, s, erep,
               ln_g.reshape(1, D), ln_b.reshape(1, D),
               W1, b1.reshape(1, F), W2, b2.reshape(1, D))


# speedup vs baseline: 1.7785x; 1.3758x over previous
import functools

import jax
import jax.numpy as jnp
from jax import lax
from jax.experimental import pallas as pl
from jax.experimental.pallas import tpu as pltpu
from jax.experimental.pallas import tpu_sc as plsc

N = 10000
E = 320000
D = 128
H = 8
K = 16
F = 512

NC = 2
NS = 16
NT = NC * NS
EPT = E // NT
WW = 80
NW = EPT // WW
RPT = N // NS
VPW = WW // 16

_MESH = dict(core_axis_name="c", subcore_axis_name="s", num_cores=NC,
             num_subcores=NS)


def _lane():
  return lax.iota(jnp.int32, 16)


def _c16(v):
  return jnp.full((16,), v, jnp.int32)


def _proj_body(w_ref, wf_ref, ab_ref, wha_ref, whb_ref, sc_ref):
  wh = jnp.dot(w_ref[...], wf_ref[...], preferred_element_type=jnp.float32)
  wha_ref[...] = wh[:, :D // 2]
  whb_ref[...] = wh[:, D // 2:]
  sc_ref[...] = jnp.dot(wh, ab_ref[...], preferred_element_type=jnp.float32)


def _project(w, wflat, abig):
  nb = 5
  bn = N // nb
  return pl.pallas_call(
      _proj_body,
      grid=(nb,),
      in_specs=[
          pl.BlockSpec((bn, D), lambda i: (i, 0)),
          pl.BlockSpec((D, D), lambda i: (0, 0)),
          pl.BlockSpec((D, 2 * H), lambda i: (0, 0)),
      ],
      out_specs=[
          pl.BlockSpec((bn, D // 2), lambda i: (i, 0)),
          pl.BlockSpec((bn, D // 2), lambda i: (i, 0)),
          pl.BlockSpec((bn, 2 * H), lambda i: (i, 0)),
      ],
      out_shape=[
          jax.ShapeDtypeStruct((N, D // 2), jnp.float32),
          jax.ShapeDtypeStruct((N, D // 2), jnp.float32),
          jax.ShapeDtypeStruct((N, 2 * H), jnp.float32),
      ],
  )(w, wflat, abig)


def _segmax_body(scores_hbm, srcslab_hbm, dstflat_hbm, p_out,
                 p_v, srcidx_v, dstflat_v, rows0_v, rows1_v, sem0, sem1):
  t = lax.axis_index("s") * NC + lax.axis_index("c")
  pltpu.sync_copy(srcslab_hbm.at[t], srcidx_v)
  pltpu.sync_copy(dstflat_hbm.at[t], dstflat_v)

  def init(i, _):
    p_v[pl.ds(i * 16, 16)] = jnp.full((16,), -1e30, jnp.float32)
    return 0
  lax.fori_loop(0, (N * H) // 16, init, 0)

  lane = _lane()
  lane8 = lane & 7
  mask8 = lane < H

  def compute(w, rows):
    def grp(g, _):
      dst16 = dstflat_v[pl.ds(w * WW + g * 16, 16)]
      for j in range(16):
        ei = g * 16 + j
        srow = rows[ei, :]
        dstv = dst16.at[_c16(j)].get(mode="promise_in_bounds")
        adr = dstv * H + lane8
        old = plsc.load_gather(p_v, [adr])
        plsc.store_scatter(p_v, [adr], jnp.maximum(old, srow), mask=mask8)
      return 0
    lax.fori_loop(0, VPW, grp, 0)

  pltpu.async_copy(scores_hbm.at[srcidx_v.at[0]], rows0_v, sem0)

  def pair(wp, _):
    wa = 2 * wp
    pltpu.make_async_copy(scores_hbm.at[srcidx_v.at[wa]], rows0_v, sem0).wait()
    pltpu.async_copy(scores_hbm.at[srcidx_v.at[wa + 1]], rows1_v, sem1)
    compute(wa, rows0_v)
    pltpu.make_async_copy(scores_hbm.at[srcidx_v.at[wa + 1]], rows1_v,
                          sem1).wait()
    pltpu.async_copy(scores_hbm.at[srcidx_v.at[wa + 2]], rows0_v, sem0)
    compute(wa + 1, rows1_v)
    return 0
  lax.fori_loop(0, (NW - 1) // 2, pair, 0)
  pltpu.make_async_copy(scores_hbm.at[srcidx_v.at[NW - 1]], rows0_v,
                        sem0).wait()
  compute(NW - 1, rows0_v)

  pltpu.sync_copy(p_v, p_out.at[t])


def _segmax(scores, srcslab, dstflat):
  fn = pl.kernel(
      _segmax_body,
      out_type=jax.ShapeDtypeStruct((NT, N * H), jnp.float32),
      mesh=plsc.VectorSubcoreMesh(**_MESH),
      compiler_params=pltpu.CompilerParams(needs_layout_passes=False, use_tc_tiling_on_sc=False),
      scratch_types=[
          pltpu.VMEM((N * H,), jnp.float32),
          pltpu.VMEM((NW, WW), jnp.int32),
          pltpu.VMEM((EPT,), jnp.int32),
          pltpu.VMEM((WW, 2 * H), jnp.float32),
          pltpu.VMEM((WW, 2 * H), jnp.float32),
          pltpu.SemaphoreType.DMA,
          pltpu.SemaphoreType.DMA,
      ],
  )
  return fn(scores, srcslab, dstflat)


def _pmax_body(p_ref, out_ref):
  out_ref[...] = jnp.max(p_ref[...], axis=0, keepdims=True)


def _pmax(p_part):
  nb = 5
  bc = (N * H) // nb
  out = pl.pallas_call(
      _pmax_body,
      grid=(nb,),
      in_specs=[pl.BlockSpec((NT, bc), lambda i: (0, i))],
      out_specs=pl.BlockSpec((1, bc), lambda i: (0, i)),
      out_shape=jax.ShapeDtypeStruct((1, N * H), jnp.float32),
  )(p_part)
  return out.reshape(N, H)


def _edm_body(p_ref, sc_ref, out_ref):
  ed = sc_ref[:, H:]
  z = p_ref[...] + ed
  m = jnp.where(z > 0, z, 0.2 * z)
  out_ref[...] = jnp.concatenate([ed, m], axis=1)


def _edm(p2, scores):
  nb = 5
  bn = N // nb
  return pl.pallas_call(
      _edm_body,
      grid=(nb,),
      in_specs=[
          pl.BlockSpec((bn, H), lambda i: (i, 0)),
          pl.BlockSpec((bn, 2 * H), lambda i: (i, 0)),
      ],
      out_specs=pl.BlockSpec((bn, 2 * H), lambda i: (i, 0)),
      out_shape=jax.ShapeDtypeStruct((N, 2 * H), jnp.float32),
  )(p2, scores)


CH = 128
MAXNZ = EPT + CH


def _edge_body(wha_hbm, whb_hbm, scores_hbm, edm_hbm, zero64_hbm, zero16_hbm,
               srcslab_hbm, dstslabr_hbm, sdflat_hbm,
               msga_out, msgb_out, den_out,
               srcidx_v, dstidxr_v, sdflat_v, nzid_v,
               srclist_v, dstlist_v,
               sr0_v, sr1_v, ed0_v, ed1_v, ex0_v, ex1_v,
               whc_v, sc2_v, ed2_v,
               msg_s, den_s, sem1, sem2, sem3, sem4):
  cid = lax.axis_index("c")
  sid = lax.axis_index("s")
  t = sid * NC + cid
  pltpu.sync_copy(srcslab_hbm.at[t], srcidx_v)
  pltpu.sync_copy(dstslabr_hbm.at[t], dstidxr_v)
  pltpu.sync_copy(sdflat_hbm.at[t], sdflat_v)
  pltpu.sync_copy(zero16_hbm, den_s.at[pl.ds(sid * RPT, RPT)])
  plsc.subcore_barrier()

  lane = _lane()
  midx = (lane & 7) + H
  mask8 = lane < H

  mask1 = lane < 1

  def issue(w, sr, ed, ss, se):
    pltpu.async_copy(scores_hbm.at[srcidx_v.at[w]], sr, ss)
    pltpu.async_copy(edm_hbm.at[dstidxr_v.at[w]], ed, se)

  def wait(w, sr, ed, ss, se):
    pltpu.make_async_copy(scores_hbm.at[srcidx_v.at[w]], sr, ss).wait()
    pltpu.make_async_copy(edm_hbm.at[dstidxr_v.at[w]], ed, se).wait()

  def compute(w, srcrows, edmrows, exbuf, ctr):
    def grp(g, ctr):
      for j in range(4):
        ei = g * 4 + j
        srow = srcrows[ei, :]
        erow = edmrows[ei, :]
        mvec = erow.at[midx].get(mode="promise_in_bounds")
        z = srow + erow
        e = jnp.where(z > 0, z, 0.2 * z)
        ex = jnp.where(mask8, jnp.exp(jnp.where(mask8, e - mvec, -1e30)), 0.0)
        exbuf[ei, :] = ex
        plsc.store_scatter(nzid_v, [_c16(ctr)], _c16(w * WW + ei), mask=mask1)
        nzc = plsc.all_reduce_population_count(ex > 0.0)
        ctr = ctr + jnp.minimum(nzc[0], 1)
      return ctr
    ctr = lax.fori_loop(0, WW // 4, grp, ctr)
    pltpu.sync_copy(exbuf, den_s.at[dstidxr_v.at[w]], add=True)
    return ctr

  issue(0, sr0_v, ed0_v, sem1, sem2)

  def pair(wp, ctr):
    wa = 2 * wp
    wait(wa, sr0_v, ed0_v, sem1, sem2)
    issue(wa + 1, sr1_v, ed1_v, sem3, sem4)
    ctr = compute(wa, sr0_v, ed0_v, ex0_v, ctr)
    wait(wa + 1, sr1_v, ed1_v, sem3, sem4)
    issue(wa + 2, sr0_v, ed0_v, sem1, sem2)
    ctr = compute(wa + 1, sr1_v, ed1_v, ex1_v, ctr)
    return ctr
  cnt = lax.fori_loop(0, (NW - 1) // 2, pair, 0)
  wait(NW - 1, sr0_v, ed0_v, sem1, sem2)
  cnt = compute(NW - 1, sr0_v, ed0_v, ex0_v, cnt)

  plsc.subcore_barrier()
  pltpu.sync_copy(den_s.at[pl.ds(sid * RPT, RPT)],
                  den_out.at[cid, pl.ds(sid * RPT, RPT)])

  nchunks = (cnt + CH - 1) // CH

  for hp, (wh_hbm, m_out) in enumerate(((wha_hbm, msga_out),
                                        (whb_hbm, msgb_out))):
    pltpu.sync_copy(zero64_hbm, msg_s.at[pl.ds(sid * RPT, RPT)])
    plsc.subcore_barrier()

    def chunk(c, _):
      for v in range(CH // 16):
        j16 = _c16(c * CH + v * 16) + lane
        valid = j16 < _c16(cnt)
        eids = nzid_v[pl.ds(c * CH + v * 16, 16)]
        eids = jnp.where(valid, eids, _c16(0))
        srcs = plsc.load_gather(sdflat_v, [eids])
        dsts = plsc.load_gather(sdflat_v, [eids + _c16(EPT)])
        dsts = jnp.where(valid, dsts, _c16(N) + (j16 & 7))
        srclist_v[pl.ds(v * 16, 16)] = srcs
        dstlist_v[pl.ds(v * 16, 16)] = dsts
      g1 = pltpu.async_copy(wh_hbm.at[srclist_v], whc_v, sem3)
      g2 = pltpu.async_copy(scores_hbm.at[srclist_v], sc2_v, sem1)
      g3 = pltpu.async_copy(edm_hbm.at[dstlist_v], ed2_v, sem2)
      g1.wait()
      g2.wait()
      g3.wait()

      def row(r, _):
        srow = sc2_v[r, :]
        erow = ed2_v[r, :]
        mvec = erow.at[midx].get(mode="promise_in_bounds")
        z = srow + erow
        e = jnp.where(z > 0, z, 0.2 * z)
        ex = jnp.where(mask8, jnp.exp(jnp.where(mask8, e - mvec, -1e30)), 0.0)
        for h in range(H // 2):
          exh = ex.at[_c16(hp * (H // 2) + h)].get(mode="promise_in_bounds")
          seg = whc_v[r, pl.ds(h * K, K)]
          whc_v[r, pl.ds(h * K, K)] = seg * exh
        return 0
      lax.fori_loop(0, CH, row, 0)

      pltpu.sync_copy(whc_v, msg_s.at[dstlist_v], add=True)
      return 0
    lax.fori_loop(0, nchunks, chunk, 0)

    plsc.subcore_barrier()
    pltpu.sync_copy(msg_s.at[pl.ds(sid * RPT, RPT)],
                    m_out.at[cid, pl.ds(sid * RPT, RPT)])
    plsc.subcore_barrier()


def _edge_phase(wha, whb, scores, edm, zero64, zero16, srcslab, dstslabr,
                sdflat):
  fn = pl.kernel(
      _edge_body,
      out_type=(
          jax.ShapeDtypeStruct((NC, N, D // 2), jnp.float32),
          jax.ShapeDtypeStruct((NC, N, D // 2), jnp.float32),
          jax.ShapeDtypeStruct((NC, N, 2 * H), jnp.float32),
      ),
      mesh=plsc.VectorSubcoreMesh(**_MESH),
      compiler_params=pltpu.CompilerParams(needs_layout_passes=False, use_tc_tiling_on_sc=False),
      scratch_types=[
          pltpu.VMEM((NW, WW), jnp.int32),
          pltpu.VMEM((NW, WW), jnp.int32),
          pltpu.VMEM((2 * EPT,), jnp.int32),
          pltpu.VMEM((MAXNZ,), jnp.int32),
          pltpu.VMEM((CH,), jnp.int32),
          pltpu.VMEM((CH,), jnp.int32),
          pltpu.VMEM((WW, 2 * H), jnp.float32),
          pltpu.VMEM((WW, 2 * H), jnp.float32),
          pltpu.VMEM((WW, 2 * H), jnp.float32),
          pltpu.VMEM((WW, 2 * H), jnp.float32),
          pltpu.VMEM((WW, 2 * H), jnp.float32),
          pltpu.VMEM((WW, 2 * H), jnp.float32),
          pltpu.VMEM((CH, D // 2), jnp.float32),
          pltpu.VMEM((CH, 2 * H), jnp.float32),
          pltpu.VMEM((CH, 2 * H), jnp.float32),
          pltpu.VMEM_SHARED((N + 8, D // 2), jnp.float32),
          pltpu.VMEM_SHARED((N + 8, 2 * H), jnp.float32),
          pltpu.SemaphoreType.DMA,
          pltpu.SemaphoreType.DMA,
          pltpu.SemaphoreType.DMA,
          pltpu.SemaphoreType.DMA,
      ],
  )
  return fn(wha, whb, scores, edm, zero64, zero16, srcslab, dstslabr,
            sdflat)


def _post_body(ma_ref, mb_ref, dp_ref, s_ref, erep_ref, lng_ref, lnb_ref,
               w1_ref, b1_ref, w2_ref, b2_ref, out_ref):
  acc = jnp.concatenate([ma_ref[0] + ma_ref[1], mb_ref[0] + mb_ref[1]],
                        axis=1)
  den = dp_ref[0, :, :H] + dp_ref[1, :, :H]
  recip = 1.0 / (den + 1e-10)
  gat = acc * jnp.dot(recip, erep_ref[...], preferred_element_type=jnp.float32)
  g = jnp.where(gat > 0, gat, jnp.exp(jnp.minimum(gat, 0.0)) - 1.0)
  hh = g + s_ref[...]
  mu = jnp.mean(hh, axis=-1, keepdims=True)
  xc = hh - mu
  var = jnp.mean(xc * xc, axis=-1, keepdims=True)
  hn = xc / jnp.sqrt(var + 1e-6) * lng_ref[...] + lnb_ref[...]
  inter = jnp.dot(hn, w1_ref[...], preferred_element_type=jnp.float32)
  inter = jnp.maximum(inter + b1_ref[...], 0.0)
  out = jnp.dot(inter, w2_ref[...], preferred_element_type=jnp.float32)
  out_ref[...] = out + b2_ref[...] + hh


def _post(msga, msgb, den_part, s, erep, ln_g, ln_b, w1, b1, w2, b2):
  nb = 5
  bn = N // nb
  return pl.pallas_call(
      _post_body,
      grid=(nb,),
      in_specs=[
          pl.BlockSpec((NC, bn, D // 2), lambda i: (0, i, 0)),
          pl.BlockSpec((NC, bn, D // 2), lambda i: (0, i, 0)),
          pl.BlockSpec((NC, bn, 2 * H), lambda i: (0, i, 0)),
          pl.BlockSpec((bn, D), lambda i: (i, 0)),
          pl.BlockSpec((H, D), lambda i: (0, 0)),
          pl.BlockSpec((1, D), lambda i: (0, 0)),
          pl.BlockSpec((1, D), lambda i: (0, 0)),
          pl.BlockSpec((D, F), lambda i: (0, 0)),
          pl.BlockSpec((1, F), lambda i: (0, 0)),
          pl.BlockSpec((F, D), lambda i: (0, 0)),
          pl.BlockSpec((1, D), lambda i: (0, 0)),
      ],
      out_specs=pl.BlockSpec((bn, D), lambda i: (i, 0)),
      out_shape=jax.ShapeDtypeStruct((N, D), jnp.float32),
  )(msga, msgb, den_part, s, erep, ln_g, ln_b, w1, b1, w2, b2)


def kernel(w, s, edge_index, W, a_src, a_dst, ln_g, ln_b, W1, b1, W2, b2):
  w = w.astype(jnp.float32)
  src = edge_index[0].astype(jnp.int32)
  dst = edge_index[1].astype(jnp.int32)

  wflat = jnp.transpose(W, (1, 0, 2)).reshape(D, H * K)
  col = jnp.arange(D)
  hcol = col // K
  asrc_m = jnp.zeros((D, H), jnp.float32).at[col, hcol].set(a_src.reshape(-1))
  adst_m = jnp.zeros((D, H), jnp.float32).at[col, hcol].set(a_dst.reshape(-1))
  abig = jnp.concatenate([asrc_m, adst_m], axis=1)
  erep = jnp.repeat(jnp.eye(H, dtype=jnp.float32), K, axis=1)

  srcslab = src.reshape(NT, NW, WW)
  dstslabr = dst.reshape(NT, NW, WW)
  dstflat = dst.reshape(NT, EPT)
  sdflat = jnp.concatenate([src.reshape(NT, EPT), dst.reshape(NT, EPT)],
                           axis=1)
  zero64 = jnp.zeros((RPT, D // 2), jnp.float32)
  zero16 = jnp.zeros((RPT, 2 * H), jnp.float32)

  wha, whb, scores = _project(w, wflat, abig)
  p_part = _segmax(scores, srcslab, dstflat)
  edm = _edm(_pmax(p_part), scores)
  msga, msgb, den_part = _edge_phase(wha, whb, scores, edm, zero64, zero16,
                                     srcslab, dstslabr, sdflat)
  return _post(msga, msgb, den_part, s, erep,
               ln_g.reshape(1, D), ln_b.reshape(1, D),
               W1, b1.reshape(1, F), W2, b2.reshape(1, D))
